# baseline (speedup 1.0000x reference)
import functools

import jax
import jax.numpy as jnp
from jax import lax
from jax.experimental import pallas as pl
from jax.experimental.pallas import tpu as pltpu
from jax.experimental.pallas import tpu_sc as plsc

N = 16384
W = 16
CH = N // W
CV = CH // 16
NB = 256
BPW = NB // W
NG = 4
PAIRS = [(0, 1), (0, 2), (0, 3), (1, 2), (1, 3), (2, 3)]
SIGN = jnp.int32(-2147483648)

_mesh = plsc.VectorSubcoreMesh(
    core_axis_name="c", subcore_axis_name="s", num_cores=1)


@functools.partial(
    pl.kernel,
    out_type=jax.ShapeDtypeStruct((16,), jnp.float32),
    mesh=_mesh,
    scratch_types=[
        pltpu.VMEM_SHARED((N,), jnp.int32),
        pltpu.VMEM_SHARED((N,), jnp.int32),
        pltpu.VMEM_SHARED((N,), jnp.int32),
        pltpu.VMEM_SHARED((N,), jnp.int32),
        pltpu.VMEM_SHARED((W, NB), jnp.int32),
        pltpu.VMEM_SHARED((W, NB), jnp.int32),
        pltpu.VMEM_SHARED((NB,), jnp.int32),
        pltpu.VMEM_SHARED((W, 16), jnp.int32),
        pltpu.VMEM_SHARED((W, 16), jnp.float32),
        pltpu.VMEM((CH,), jnp.float32),
        pltpu.VMEM((CH,), jnp.int32),
        pltpu.VMEM((CH,), jnp.int32),
        pltpu.VMEM((8, 128), jnp.int32),
        pltpu.VMEM((NB,), jnp.int32),
        pltpu.VMEM((W, 16), jnp.int32),
        pltpu.VMEM((NB,), jnp.int32),
        pltpu.VMEM((NB,), jnp.int32),
        pltpu.VMEM((W, 16), jnp.float32),
        pltpu.VMEM((16,), jnp.float32),
        pltpu.SemaphoreType.DMA,
    ],
    compiler_params=pltpu.CompilerParams(needs_layout_passes=False),
)
def _wd_kernel(pred_hbm, grp_hbm, out_hbm,
               sp_key0, sp_key1, sp_grp0, sp_grp1,
               sp_hist, sp_pre, sp_tot, sp_gcnt, sp_acc,
               loc_f, loc_key, loc_grp, loc_dest, loc_off, loc_own,
               loc_tot, loc_pre, loc_accin, loc_out, sem):
    me = lax.axis_index("s")
    base = me * CH
    iota = lax.iota(jnp.int32, 16)
    ones = jnp.ones((16,), jnp.int32)

    _c1 = pltpu.async_copy(pred_hbm.at[pl.ds(base, CH)], loc_f, sem)
    _c2 = pltpu.async_copy(grp_hbm.at[pl.ds(base, CH)], loc_grp, sem)
    _c1.wait(); _c2.wait()
    for j in range(NB // 16):
        loc_off[pl.ds(j * 16, 16)] = jnp.zeros((16,), jnp.int32)

    def conv(i, c):
        f = loc_f[pl.ds(i * 16, 16)]
        bits = plsc.bitcast(f, jnp.int32)
        sgn = lax.shift_right_arithmetic(bits, 31)
        k = bits ^ (sgn | SIGN)
        loc_key[pl.ds(i * 16, 16)] = k
        plsc.addupdate_scatter(loc_off, [k & 255], ones)
        return c

    lax.fori_loop(0, CV, conv, jnp.int32(0), unroll=4)
    pltpu.sync_copy(loc_off, sp_hist.at[me])
    plsc.subcore_barrier()

    bufs = [(sp_key0, sp_grp0, sp_key1, sp_grp1),
            (sp_key1, sp_grp1, sp_key0, sp_grp0)]
    for p in range(4):
        sk, sg, dk, dg = bufs[p % 2]
        shift = 8 * p
        if p > 0:
            _c1 = pltpu.async_copy(sk.at[pl.ds(base, CH)], loc_key, sem)
            _c2 = pltpu.async_copy(sg.at[pl.ds(base, CH)], loc_grp, sem)
            _c1.wait(); _c2.wait()
            for j in range(NB // 16):
                loc_off[pl.ds(j * 16, 16)] = jnp.zeros((16,), jnp.int32)

            def hb(i, c, shift=shift):
                k = loc_key[pl.ds(i * 16, 16)]
                d = lax.shift_right_logical(k, shift) & 255
                plsc.addupdate_scatter(loc_off, [d], ones)
                return c

            lax.fori_loop(0, CV, hb, jnp.int32(0), unroll=4)
            pltpu.sync_copy(loc_off, sp_hist.at[me])
            plsc.subcore_barrier()

        col = me * BPW
        cps = [pltpu.async_copy(sp_hist.at[t, pl.ds(col, BPW)],
                                loc_own.at[t], sem) for t in range(W)]
        for cp in cps:
            cp.wait()
        run = jnp.zeros((16,), jnp.int32)
        for t in range(W):
            h = loc_own[t, :]
            loc_own[t, :] = run
            run = run + h
        loc_tot[pl.ds(0, 16)] = run
        cps = [pltpu.async_copy(loc_own.at[t], sp_pre.at[t, pl.ds(col, BPW)],
                                sem) for t in range(W)]
        cps.append(pltpu.async_copy(loc_tot.at[pl.ds(0, 16)],
                                    sp_tot.at[pl.ds(col, BPW)], sem))
        for cp in cps:
            cp.wait()
        plsc.subcore_barrier()

        _c1 = pltpu.async_copy(sp_tot, loc_tot, sem)
        _c2 = pltpu.async_copy(sp_pre.at[me], loc_pre, sem)
        _c1.wait(); _c2.wait()

        def ob(j, carry):
            h = loc_tot[pl.ds(j * 16, 16)]
            cs = plsc.cumsum(h)
            loc_off[pl.ds(j * 16, 16)] = (cs - h + carry
                                          + loc_pre[pl.ds(j * 16, 16)])
            return carry + cs.at[jnp.full((16,), 15, jnp.int32)].get(
                mode='promise_in_bounds')

        lax.fori_loop(0, NB // 16, ob, jnp.zeros((16,), jnp.int32), unroll=4)

        copies = []
        for r in range(8):
            def sb(k2, c, r=r, shift=shift):
                i = r * 8 + k2
                k = loc_key[pl.ds(i * 16, 16)]
                d = lax.shift_right_logical(k, shift) & 255
                occ, _last = plsc.scan_count(d)
                bs = plsc.load_gather(loc_off, [d])
                loc_dest[r, pl.ds(k2 * 16, 16)] = bs + occ - 1
                plsc.addupdate_scatter(loc_off, [d], ones)
                return c

            lax.fori_loop(0, 8, sb, jnp.int32(0), unroll=4)
            copies.append(pltpu.async_copy(
                loc_key.at[pl.ds(r * 128, 128)], dk.at[loc_dest.at[r]], sem))
            copies.append(pltpu.async_copy(
                loc_grp.at[pl.ds(r * 128, 128)], dg.at[loc_dest.at[r]], sem))
        for cp in copies:
            cp.wait()
        plsc.subcore_barrier()

    _c1 = pltpu.async_copy(sp_key0.at[pl.ds(base, CH)], loc_key, sem)
    _c2 = pltpu.async_copy(sp_grp0.at[pl.ds(base, CH)], loc_grp, sem)
    _c1.wait(); _c2.wait()

    def cnt(i, t):
        g = loc_grp[pl.ds(i * 16, 16)]
        return tuple(t[q] + (g == q).astype(jnp.int32) for q in range(NG))

    tv = lax.fori_loop(0, CV, cnt,
                       tuple(jnp.zeros((16,), jnp.int32) for _ in range(NG)),
                       unroll=4)
    grow = jnp.zeros((16,), jnp.int32)
    for q in range(NG):
        grow = jnp.where(iota == q,
                         jnp.full((16,), jnp.sum(tv[q]), jnp.int32), grow)
    loc_own[0, :] = grow
    pltpu.sync_copy(loc_own.at[0], sp_gcnt.at[me])
    plsc.subcore_barrier()
    pltpu.sync_copy(sp_gcnt, loc_own)

    def gp(t, a):
        return a + loc_own[t, :]

    pre_vec = lax.fori_loop(0, me, gp, jnp.zeros((16,), jnp.int32))
    tot_vec = lax.fori_loop(0, W, gp, jnp.zeros((16,), jnp.int32))

    def lane_bcast(vec, q):
        return jnp.full((16,), jnp.sum(jnp.where(iota == q, vec, 0)),
                        jnp.int32)

    nstart = [lane_bcast(pre_vec, q) for q in range(NG)]
    totals = [lane_bcast(tot_vec, q) for q in range(NG)]
    mvec = [jnp.minimum(totals[a], totals[b]) for a, b in PAIRS]

    dcar0 = []
    for t6, (a, b) in enumerate(PAIRS):
        m = mvec[t6]
        d0 = jnp.minimum(nstart[a], m) - jnp.minimum(nstart[b], m)
        dcar0.append(jnp.abs(d0).astype(jnp.float32))

    lane0 = iota == 0
    prev_idx = jnp.maximum(iota - 1, 0)
    last_idx = jnp.full((16,), 15, jnp.int32)

    def scb(i, carry):
        n = carry[:NG]
        dcar = carry[NG:NG + 6]
        accs = carry[NG + 6:]
        k = loc_key[pl.ds(i * 16, 16)]
        g = loc_grp[pl.ds(i * 16, 16)]
        sgn2 = lax.shift_right_arithmetic(k, 31)
        v = plsc.bitcast(k ^ ((~sgn2) | SIGN), jnp.float32)
        c, nn = [], []
        for q in range(NG):
            cq = n[q] + plsc.cumsum((g == q).astype(jnp.int32))
            c.append(cq)
            nn.append(cq.at[last_idx].get(mode='promise_in_bounds'))
        ncar, na = [], []
        for t6, (a, b) in enumerate(PAIRS):
            m = mvec[t6]
            d = jnp.minimum(c[a], m) - jnp.minimum(c[b], m)
            dd = jnp.abs(d).astype(jnp.float32)
            dsh = jnp.where(
                lane0, dcar[t6],
                dd.at[prev_idx].get(mode='promise_in_bounds'))
            ncar.append(dd.at[last_idx].get(mode='promise_in_bounds'))
            na.append(accs[t6] + v * (dsh - dd))
        return tuple(nn) + tuple(ncar) + tuple(na)

    init = (tuple(nstart) + tuple(dcar0)
            + tuple(jnp.zeros((16,), jnp.float32) for _ in range(6)))
    fin = lax.fori_loop(0, CV, scb, init, unroll=4)
    accs = fin[NG + 6:]
    arow = jnp.zeros((16,), jnp.float32)
    for t6 in range(len(PAIRS)):
        arow = jnp.where(iota == t6,
                         jnp.full((16,), jnp.sum(accs[t6]), jnp.float32), arow)
    loc_out[...] = arow
    pltpu.sync_copy(loc_out, sp_acc.at[me])
    plsc.subcore_barrier()

    @pl.when(me == 0)
    def _fin():
        pltpu.sync_copy(sp_acc, loc_accin)
        s = jnp.zeros((16,), jnp.float32)
        for t in range(W):
            s = s + loc_accin[t, :]
        mv6 = jnp.ones((16,), jnp.float32)
        for t6, (a, b) in enumerate(PAIRS):
            mv6 = jnp.where(iota == t6,
                            jnp.minimum(totals[a], totals[b])
                            .astype(jnp.float32), mv6)
        wd = s / mv6
        res = jnp.sum(jnp.where(iota < 6, wd, jnp.float32(0.0)))
        loc_out[...] = (jnp.full((16,), res, jnp.float32)
                        * jnp.float32(1.0 / len(PAIRS)))
        pltpu.sync_copy(loc_out, out_hbm)


def kernel(batch_pred, batch_group):
    out = _wd_kernel(batch_pred, batch_group.astype(jnp.int32))
    return out[0]


# speedup vs baseline: 1.0029x; 1.0029x over previous
import functools

import jax
import jax.numpy as jnp
from jax import lax
from jax.experimental import pallas as pl
from jax.experimental.pallas import tpu as pltpu
from jax.experimental.pallas import tpu_sc as plsc

N = 16384
W = 16
CH = N // W
CV = CH // 16
NB = 256
BPW = NB // W
NG = 4
PAIRS = [(0, 1), (0, 2), (0, 3), (1, 2), (1, 3), (2, 3)]
SIGN = jnp.int32(-2147483648)

_mesh = plsc.VectorSubcoreMesh(
    core_axis_name="c", subcore_axis_name="s", num_cores=1)


@functools.partial(
    pl.kernel,
    out_type=jax.ShapeDtypeStruct((16,), jnp.float32),
    mesh=_mesh,
    scratch_types=[
        pltpu.VMEM_SHARED((N,), jnp.int32),
        pltpu.VMEM_SHARED((N,), jnp.int32),
        pltpu.VMEM_SHARED((N,), jnp.int32),
        pltpu.VMEM_SHARED((N,), jnp.int32),
        pltpu.VMEM_SHARED((W, NB), jnp.int32),
        pltpu.VMEM_SHARED((W, NB), jnp.int32),
        pltpu.VMEM_SHARED((NB,), jnp.int32),
        pltpu.VMEM_SHARED((W, 16), jnp.int32),
        pltpu.VMEM_SHARED((W, 16), jnp.float32),
        pltpu.VMEM((CH,), jnp.float32),
        pltpu.VMEM((CH,), jnp.int32),
        pltpu.VMEM((CH,), jnp.int32),
        pltpu.VMEM((8, 128), jnp.int32),
        pltpu.VMEM((NB,), jnp.int32),
        pltpu.VMEM((W, 16), jnp.int32),
        pltpu.VMEM((NB,), jnp.int32),
        pltpu.VMEM((NB,), jnp.int32),
        pltpu.VMEM((W, 16), jnp.float32),
        pltpu.VMEM((16,), jnp.float32),
        pltpu.SemaphoreType.DMA,
    ],
    compiler_params=pltpu.CompilerParams(needs_layout_passes=False),
)
def _wd_kernel(pred_hbm, grp_hbm, out_hbm,
               sp_key0, sp_key1, sp_grp0, sp_grp1,
               sp_hist, sp_pre, sp_tot, sp_gcnt, sp_acc,
               loc_f, loc_key, loc_grp, loc_dest, loc_off, loc_own,
               loc_tot, loc_pre, loc_accin, loc_out, sem):
    me = lax.axis_index("s")
    base = me * CH
    iota = lax.iota(jnp.int32, 16)
    ones = jnp.ones((16,), jnp.int32)

    _c1 = pltpu.async_copy(pred_hbm.at[pl.ds(base, CH)], loc_f, sem)
    _c2 = pltpu.async_copy(grp_hbm.at[pl.ds(base, CH)], loc_grp, sem)
    _c1.wait(); _c2.wait()
    for j in range(NB // 16):
        loc_off[pl.ds(j * 16, 16)] = jnp.zeros((16,), jnp.int32)

    def conv(i, c):
        f = loc_f[pl.ds(i * 16, 16)]
        bits = plsc.bitcast(f, jnp.int32)
        sgn = lax.shift_right_arithmetic(bits, 31)
        k = bits ^ (sgn | SIGN)
        loc_key[pl.ds(i * 16, 16)] = k
        plsc.addupdate_scatter(loc_off, [k & 255], ones)
        return c

    lax.fori_loop(0, CV, conv, jnp.int32(0), unroll=4)
    pltpu.sync_copy(loc_off, sp_hist.at[me])
    plsc.subcore_barrier()

    bufs = [(sp_key0, sp_grp0, sp_key1, sp_grp1),
            (sp_key1, sp_grp1, sp_key0, sp_grp0)]
    for p in range(4):
        sk, sg, dk, dg = bufs[p % 2]
        shift = 8 * p
        if p > 0:
            _c1 = pltpu.async_copy(sk.at[pl.ds(base, CH)], loc_key, sem)
            _c2 = pltpu.async_copy(sg.at[pl.ds(base, CH)], loc_grp, sem)
            _c1.wait(); _c2.wait()
            for j in range(NB // 16):
                loc_off[pl.ds(j * 16, 16)] = jnp.zeros((16,), jnp.int32)

            def hb(i, c, shift=shift):
                k = loc_key[pl.ds(i * 16, 16)]
                d = lax.shift_right_logical(k, shift) & 255
                plsc.addupdate_scatter(loc_off, [d], ones)
                return c

            lax.fori_loop(0, CV, hb, jnp.int32(0), unroll=4)
            pltpu.sync_copy(loc_off, sp_hist.at[me])
            plsc.subcore_barrier()

        col = me * BPW
        cps = [pltpu.async_copy(sp_hist.at[t, pl.ds(col, BPW)],
                                loc_own.at[t], sem) for t in range(W)]
        for cp in cps:
            cp.wait()
        run = jnp.zeros((16,), jnp.int32)
        for t in range(W):
            h = loc_own[t, :]
            loc_own[t, :] = run
            run = run + h
        loc_tot[pl.ds(0, 16)] = run
        cps = [pltpu.async_copy(loc_own.at[t], sp_pre.at[t, pl.ds(col, BPW)],
                                sem) for t in range(W)]
        cps.append(pltpu.async_copy(loc_tot.at[pl.ds(0, 16)],
                                    sp_tot.at[pl.ds(col, BPW)], sem))
        for cp in cps:
            cp.wait()
        plsc.subcore_barrier()

        _c1 = pltpu.async_copy(sp_tot, loc_tot, sem)
        _c2 = pltpu.async_copy(sp_pre.at[me], loc_pre, sem)
        _c1.wait(); _c2.wait()

        def ob(j, carry):
            h = loc_tot[pl.ds(j * 16, 16)]
            cs = plsc.cumsum(h)
            loc_off[pl.ds(j * 16, 16)] = (cs - h + carry
                                          + loc_pre[pl.ds(j * 16, 16)])
            return carry + cs.at[jnp.full((16,), 15, jnp.int32)].get(
                mode='promise_in_bounds')

        lax.fori_loop(0, NB // 16, ob, jnp.zeros((16,), jnp.int32), unroll=2)

        copies = []
        for r in range(8):
            def sb(k2, c, r=r, shift=shift):
                i = r * 8 + k2
                k = loc_key[pl.ds(i * 16, 16)]
                d = lax.shift_right_logical(k, shift) & 255
                occ, _last = plsc.scan_count(d)
                bs = plsc.load_gather(loc_off, [d])
                loc_dest[r, pl.ds(k2 * 16, 16)] = bs + occ - 1
                plsc.addupdate_scatter(loc_off, [d], ones)
                return c

            lax.fori_loop(0, 8, sb, jnp.int32(0), unroll=2)
            copies.append(pltpu.async_copy(
                loc_key.at[pl.ds(r * 128, 128)], dk.at[loc_dest.at[r]], sem))
            copies.append(pltpu.async_copy(
                loc_grp.at[pl.ds(r * 128, 128)], dg.at[loc_dest.at[r]], sem))
        for cp in copies:
            cp.wait()
        plsc.subcore_barrier()

    _c1 = pltpu.async_copy(sp_key0.at[pl.ds(base, CH)], loc_key, sem)
    _c2 = pltpu.async_copy(sp_grp0.at[pl.ds(base, CH)], loc_grp, sem)
    _c1.wait(); _c2.wait()

    def cnt(i, t):
        g = loc_grp[pl.ds(i * 16, 16)]
        return tuple(t[q] + (g == q).astype(jnp.int32) for q in range(NG))

    tv = lax.fori_loop(0, CV, cnt,
                       tuple(jnp.zeros((16,), jnp.int32) for _ in range(NG)),
                       unroll=4)
    grow = jnp.zeros((16,), jnp.int32)
    for q in range(NG):
        grow = jnp.where(iota == q,
                         jnp.full((16,), jnp.sum(tv[q]), jnp.int32), grow)
    loc_own[0, :] = grow
    pltpu.sync_copy(loc_own.at[0], sp_gcnt.at[me])
    plsc.subcore_barrier()
    pltpu.sync_copy(sp_gcnt, loc_own)

    def gp(t, a):
        return a + loc_own[t, :]

    pre_vec = lax.fori_loop(0, me, gp, jnp.zeros((16,), jnp.int32))
    tot_vec = lax.fori_loop(0, W, gp, jnp.zeros((16,), jnp.int32))

    def lane_bcast(vec, q):
        return jnp.full((16,), jnp.sum(jnp.where(iota == q, vec, 0)),
                        jnp.int32)

    nstart = [lane_bcast(pre_vec, q) for q in range(NG)]
    totals = [lane_bcast(tot_vec, q) for q in range(NG)]
    mvec = [jnp.minimum(totals[a], totals[b]) for a, b in PAIRS]

    dcar0 = []
    for t6, (a, b) in enumerate(PAIRS):
        m = mvec[t6]
        d0 = jnp.minimum(nstart[a], m) - jnp.minimum(nstart[b], m)
        dcar0.append(jnp.abs(d0).astype(jnp.float32))

    lane0 = iota == 0
    prev_idx = jnp.maximum(iota - 1, 0)
    last_idx = jnp.full((16,), 15, jnp.int32)

    def scb(i, carry):
        n = carry[:NG]
        dcar = carry[NG:NG + 6]
        accs = carry[NG + 6:]
        k = loc_key[pl.ds(i * 16, 16)]
        g = loc_grp[pl.ds(i * 16, 16)]
        sgn2 = lax.shift_right_arithmetic(k, 31)
        v = plsc.bitcast(k ^ ((~sgn2) | SIGN), jnp.float32)
        c, nn = [], []
        for q in range(NG):
            cq = n[q] + plsc.cumsum((g == q).astype(jnp.int32))
            c.append(cq)
            nn.append(cq.at[last_idx].get(mode='promise_in_bounds'))
        ncar, na = [], []
        for t6, (a, b) in enumerate(PAIRS):
            m = mvec[t6]
            d = jnp.minimum(c[a], m) - jnp.minimum(c[b], m)
            dd = jnp.abs(d).astype(jnp.float32)
            dsh = jnp.where(
                lane0, dcar[t6],
                dd.at[prev_idx].get(mode='promise_in_bounds'))
            ncar.append(dd.at[last_idx].get(mode='promise_in_bounds'))
            na.append(accs[t6] + v * (dsh - dd))
        return tuple(nn) + tuple(ncar) + tuple(na)

    init = (tuple(nstart) + tuple(dcar0)
            + tuple(jnp.zeros((16,), jnp.float32) for _ in range(6)))
    fin = lax.fori_loop(0, CV, scb, init, unroll=2)
    accs = fin[NG + 6:]
    arow = jnp.zeros((16,), jnp.float32)
    for t6 in range(len(PAIRS)):
        arow = jnp.where(iota == t6,
                         jnp.full((16,), jnp.sum(accs[t6]), jnp.float32), arow)
    loc_out[...] = arow
    pltpu.sync_copy(loc_out, sp_acc.at[me])
    plsc.subcore_barrier()

    @pl.when(me == 0)
    def _fin():
        pltpu.sync_copy(sp_acc, loc_accin)
        s = jnp.zeros((16,), jnp.float32)
        for t in range(W):
            s = s + loc_accin[t, :]
        mv6 = jnp.ones((16,), jnp.float32)
        for t6, (a, b) in enumerate(PAIRS):
            mv6 = jnp.where(iota == t6,
                            jnp.minimum(totals[a], totals[b])
                            .astype(jnp.float32), mv6)
        wd = s / mv6
        res = jnp.sum(jnp.where(iota < 6, wd, jnp.float32(0.0)))
        loc_out[...] = (jnp.full((16,), res, jnp.float32)
                        * jnp.float32(1.0 / len(PAIRS)))
        pltpu.sync_copy(loc_out, out_hbm)


def kernel(batch_pred, batch_group):
    out = _wd_kernel(batch_pred, batch_group.astype(jnp.int32))
    return out[0]
